# baseline (speedup 1.0000x reference)
import functools
import math

import jax
import jax.numpy as jnp
from jax import lax
from jax.experimental import pallas as pl
from jax.experimental.pallas import tpu as pltpu
from jax.experimental.pallas import tpu_sc as plsc

_ACT_C = 1.6790
_N = 10000
_E = 320000
_D = 128
_NA = 4
_NB = 8
_H = 64
_AVG_NEIGH = 32.0

_NCORES = 2
_NSUB = 16
_NW = _NCORES * _NSUB
_SLABS = (89600, 115200, 115200)
_C = 80
_CR = 200
_NCH_N = _N // _CR


def _silu_n(x):
    return _ACT_C * x * jax.nn.sigmoid(x)


_EBLK = 12800


def _edge_mlp_body(ee_ref, ea_ref, w0_ref, w1_ref, w2_ref, out_ref):
    bf = jnp.bfloat16
    h = _silu_n(jnp.dot(ee_ref[...].astype(bf), w0_ref[...].astype(bf),
                        preferred_element_type=jnp.float32))
    h = _silu_n(jnp.dot(h.astype(bf), w1_ref[...].astype(bf),
                        preferred_element_type=jnp.float32))
    w = jnp.dot(h.astype(bf), w2_ref[...].astype(bf),
                preferred_element_type=jnp.float32)
    out_ref[...] = w * ea_ref[...]


def _edge_mlp(ee, ea, w0, w1, w2):
    return pl.pallas_call(
        _edge_mlp_body,
        grid=(ee.shape[0] // _EBLK,),
        in_specs=[
            pl.BlockSpec((_EBLK, _NB), lambda i: (i, 0)),
            pl.BlockSpec((_EBLK, 1), lambda i: (i, 0)),
            pl.BlockSpec((_NB, _H), lambda i: (0, 0)),
            pl.BlockSpec((_H, _H), lambda i: (0, 0)),
            pl.BlockSpec((_H, _D), lambda i: (0, 0)),
        ],
        out_specs=pl.BlockSpec((_EBLK, _D), lambda i: (i, 0)),
        out_shape=jax.ShapeDtypeStruct((ee.shape[0], _D), jnp.float32),
    )(ee, ea, w0, w1, w2)


def _lin_body(a_ref, w_ref, o_ref):
    o_ref[...] = jnp.dot(a_ref[...], w_ref[...], preferred_element_type=jnp.float32)


def _lin1(a, w):
    blk = 2000
    return pl.pallas_call(
        _lin_body,
        grid=(_N // blk,),
        in_specs=[
            pl.BlockSpec((blk, _D), lambda i: (i, 0)),
            pl.BlockSpec((_D, _D), lambda i: (0, 0)),
        ],
        out_specs=pl.BlockSpec((blk, _D), lambda i: (i, 0)),
        out_shape=jax.ShapeDtypeStruct((_N, _D), jnp.float32),
    )(a, w)


def _sc_message_pass(x, src, dst, ew, agg_in=None):
    epw = src.shape[0] // _NW
    nchunk = epw // _C
    mesh = plsc.VectorSubcoreMesh(core_axis_name="c", subcore_axis_name="s")

    @functools.partial(
        pl.kernel,
        out_type=jax.ShapeDtypeStruct((_NCORES, _N, _D), jnp.float32),
        mesh=mesh,
        scratch_types=[
            pltpu.VMEM((2, _C), jnp.int32),
            pltpu.VMEM((2, _C), jnp.int32),
            pltpu.VMEM((2, _C, _D), jnp.float32),
            pltpu.VMEM((2, _C, _D), jnp.float32),
            pltpu.VMEM_SHARED((_N, _D), jnp.float32),
            pltpu.SemaphoreType.DMA,
            pltpu.SemaphoreType.DMA,
            pltpu.SemaphoreType.DMA,
            pltpu.SemaphoreType.DMA,
            pltpu.SemaphoreType.DMA,
            pltpu.SemaphoreType.DMA,
        ],
    )
    def k(x_hbm, src_hbm, dst_hbm, ew_hbm, *rest):
        if agg_in is not None:
            (in_hbm, out_hbm, sidx, didx, rows, ewv, acc,
             semi0, semi1, semg0, semg1, seme0, seme1) = rest
        else:
            in_hbm = None
            (out_hbm, sidx, didx, rows, ewv, acc,
             semi0, semi1, semg0, semg1, seme0, seme1) = rest
        semi = (semi0, semi1)
        semg = (semg0, semg1)
        seme = (seme0, seme1)
        cid = lax.axis_index("c")
        sid = lax.axis_index("s")
        wid = sid * _NCORES + cid
        ebase = wid * epw

        def idx_start(ch, b):
            base = ebase + ch * _C
            pltpu.async_copy(src_hbm.at[pl.ds(base, _C)], sidx.at[b], semi[b])
            pltpu.async_copy(dst_hbm.at[pl.ds(base, _C)], didx.at[b], semi[b])

        def idx_wait(ch, b):
            base = ebase + ch * _C
            pltpu.make_async_copy(src_hbm.at[pl.ds(base, _C)], sidx.at[b], semi[b]).wait()
            pltpu.make_async_copy(dst_hbm.at[pl.ds(base, _C)], didx.at[b], semi[b]).wait()

        def fetch_start(ch, b):
            base = ebase + ch * _C
            pltpu.async_copy(x_hbm.at[sidx.at[b]], rows.at[b], semg[b])
            pltpu.async_copy(ew_hbm.at[pl.ds(base, _C)], ewv.at[b], seme[b])

        def fetch_wait(ch, b):
            base = ebase + ch * _C
            pltpu.make_async_copy(x_hbm.at[sidx.at[b]], rows.at[b], semg[b]).wait()
            pltpu.make_async_copy(ew_hbm.at[pl.ds(base, _C)], ewv.at[b], seme[b]).wait()

        nzch = _N // _C

        if agg_in is None:
            rows_z = rows.at[0]

            @pl.loop(0, _C)
            def _(i):
                for g in range(_D // 16):
                    rows_z[i, pl.ds(g * 16, 16)] = jnp.zeros((16,), jnp.float32)

            @pl.loop(0, (nzch + _NSUB - 1) // _NSUB)
            def _(t):
                j = sid + t * _NSUB

                @pl.when(j < nzch)
                def _():
                    pltpu.sync_copy(rows_z, acc.at[pl.ds(j * _C, _C)])
        else:
            @pl.loop(0, (_NCH_N + _NSUB - 1) // _NSUB)
            def _(t):
                j = sid + t * _NSUB

                @pl.when(j < _NCH_N)
                def _():
                    pltpu.sync_copy(in_hbm.at[cid].at[pl.ds(j * _CR, _CR)],
                                    acc.at[pl.ds(j * _CR, _CR)])

        plsc.subcore_barrier()

        idx_start(0, 0)
        idx_start(1, 1)
        idx_wait(0, 0)
        fetch_start(0, 0)

        @pl.loop(0, (nchunk + 1) // 2)
        def _(p):
            for kk in range(2):
                ch = p * 2 + kk
                b = kk
                nb = 1 - kk

                @pl.when(ch < nchunk)
                def _():
                    @pl.when(ch + 1 < nchunk)
                    def _():
                        idx_wait(ch + 1, nb)
                        fetch_start(ch + 1, nb)

                    fetch_wait(ch, b)
                    rows_b = rows.at[b]
                    ewv_b = ewv.at[b]

                    @pl.loop(0, _C, step=4)
                    def _(r0):
                        for dr in range(4):
                            for g in range(_D // 16):
                                s = pl.ds(g * 16, 16)
                                rows_b[r0 + dr, s] = rows_b[r0 + dr, s] * ewv_b[r0 + dr, s]

                    pltpu.sync_copy(rows_b, acc.at[didx.at[b]], add=True)

                    @pl.when(ch + 2 < nchunk)
                    def _():
                        idx_start(ch + 2, b)

        plsc.subcore_barrier()

        @pl.loop(0, (_NCH_N + _NSUB - 1) // _NSUB)
        def _(t):
            j = sid + t * _NSUB

            @pl.when(j < _NCH_N)
            def _():
                pltpu.sync_copy(acc.at[pl.ds(j * _CR, _CR)],
                                out_hbm.at[cid].at[pl.ds(j * _CR, _CR)])

    if agg_in is not None:
        return k(x, src, dst, ew, agg_in)
    return k(x, src, dst, ew)


def _final_body(nf_ref, na_ref, agg_ref, wl2_ref, wsc_ref, o_ref):
    agg = agg_ref[0] + agg_ref[1]
    y = jnp.dot(agg, wl2_ref[...], preferred_element_type=jnp.float32)
    nf = nf_ref[...]
    for v in range(_NA):
        y = y + jnp.dot(nf, wsc_ref[v], preferred_element_type=jnp.float32) * na_ref[:, v:v + 1]
    o_ref[...] = nf + _silu_n(y)


def _final(nf, na, agg2, wl2, wsc):
    blk = 2000
    return pl.pallas_call(
        _final_body,
        grid=(_N // blk,),
        in_specs=[
            pl.BlockSpec((blk, _D), lambda i: (i, 0)),
            pl.BlockSpec((blk, _NA), lambda i: (i, 0)),
            pl.BlockSpec((_NCORES, blk, _D), lambda i: (0, i, 0)),
            pl.BlockSpec((_D, _D), lambda i: (0, 0)),
            pl.BlockSpec((_NA, _D, _D), lambda i: (0, 0, 0)),
        ],
        out_specs=pl.BlockSpec((blk, _D), lambda i: (i, 0)),
        out_shape=jax.ShapeDtypeStruct((_N, _D), jnp.float32),
    )(nf, na, agg2, wl2, wsc)


def kernel(node_features, edge_index, edge_attrs, edge_embeddings, node_attrs,
           W_lin1, W_fc0, W_fc1, W_fc2, W_lin2, W_sc):
    src = edge_index[0]
    dst = edge_index[1]

    w0 = W_fc0 / math.sqrt(_NB)
    w1 = W_fc1 / math.sqrt(_H)
    w2 = W_fc2 / math.sqrt(_H)

    x = _lin1(node_features, W_lin1 / math.sqrt(_D))

    agg2 = None
    lo = 0
    for n_edges in _SLABS:
        sl = slice(lo, lo + n_edges)
        lo += n_edges
        ew = _edge_mlp(edge_embeddings[sl], edge_attrs[sl], w0, w1, w2)
        agg2 = _sc_message_pass(x, src[sl], dst[sl], ew, agg2)

    wl2 = W_lin2 / (math.sqrt(_D) * math.sqrt(_AVG_NEIGH))
    wsc = jnp.transpose(W_sc, (1, 0, 2)) / math.sqrt(_D * _NA)
    return _final(node_features, node_attrs, agg2, wl2, wsc)


# speedup vs baseline: 1.0105x; 1.0105x over previous
import functools
import math

import jax
import jax.numpy as jnp
from jax import lax
from jax.experimental import pallas as pl
from jax.experimental.pallas import tpu as pltpu
from jax.experimental.pallas import tpu_sc as plsc

_ACT_C = 1.6790
_N = 10000
_E = 320000
_D = 128
_NA = 4
_NB = 8
_H = 64
_AVG_NEIGH = 32.0

_NCORES = 2
_NSUB = 16
_NW = _NCORES * _NSUB
_SLABS = (102400, 102400, 115200)
_C = 80
_CR = 200
_NCH_N = _N // _CR


def _silu_n(x):
    return _ACT_C * x * jax.nn.sigmoid(x)


_EBLK = 12800


def _edge_mlp_body(ee_ref, ea_ref, w0_ref, w1_ref, w2_ref, out_ref):
    bf = jnp.bfloat16
    h = _silu_n(jnp.dot(ee_ref[...].astype(bf), w0_ref[...].astype(bf),
                        preferred_element_type=jnp.float32))
    h = _silu_n(jnp.dot(h.astype(bf), w1_ref[...].astype(bf),
                        preferred_element_type=jnp.float32))
    w = jnp.dot(h.astype(bf), w2_ref[...].astype(bf),
                preferred_element_type=jnp.float32)
    out_ref[...] = w * ea_ref[...]


def _edge_mlp(ee, ea, w0, w1, w2):
    return pl.pallas_call(
        _edge_mlp_body,
        grid=(ee.shape[0] // _EBLK,),
        in_specs=[
            pl.BlockSpec((_EBLK, _NB), lambda i: (i, 0)),
            pl.BlockSpec((_EBLK, 1), lambda i: (i, 0)),
            pl.BlockSpec((_NB, _H), lambda i: (0, 0)),
            pl.BlockSpec((_H, _H), lambda i: (0, 0)),
            pl.BlockSpec((_H, _D), lambda i: (0, 0)),
        ],
        out_specs=pl.BlockSpec((_EBLK, _D), lambda i: (i, 0)),
        out_shape=jax.ShapeDtypeStruct((ee.shape[0], _D), jnp.float32),
    )(ee, ea, w0, w1, w2)


def _lin_body(a_ref, w_ref, o_ref):
    o_ref[...] = jnp.dot(a_ref[...], w_ref[...], preferred_element_type=jnp.float32)


def _lin1(a, w):
    blk = 2000
    return pl.pallas_call(
        _lin_body,
        grid=(_N // blk,),
        in_specs=[
            pl.BlockSpec((blk, _D), lambda i: (i, 0)),
            pl.BlockSpec((_D, _D), lambda i: (0, 0)),
        ],
        out_specs=pl.BlockSpec((blk, _D), lambda i: (i, 0)),
        out_shape=jax.ShapeDtypeStruct((_N, _D), jnp.float32),
    )(a, w)


def _sc_message_pass(x, src, dst, ew, agg_in=None):
    epw = src.shape[0] // _NW
    nchunk = epw // _C
    mesh = plsc.VectorSubcoreMesh(core_axis_name="c", subcore_axis_name="s")

    @functools.partial(
        pl.kernel,
        out_type=jax.ShapeDtypeStruct((_NCORES, _N, _D), jnp.float32),
        mesh=mesh,
        scratch_types=[
            pltpu.VMEM((2, _C), jnp.int32),
            pltpu.VMEM((2, _C), jnp.int32),
            pltpu.VMEM((2, _C, _D), jnp.float32),
            pltpu.VMEM((2, _C, _D), jnp.float32),
            pltpu.VMEM_SHARED((_N, _D), jnp.float32),
            pltpu.SemaphoreType.DMA,
            pltpu.SemaphoreType.DMA,
            pltpu.SemaphoreType.DMA,
            pltpu.SemaphoreType.DMA,
            pltpu.SemaphoreType.DMA,
            pltpu.SemaphoreType.DMA,
        ],
    )
    def k(x_hbm, src_hbm, dst_hbm, ew_hbm, *rest):
        if agg_in is not None:
            (in_hbm, out_hbm, sidx, didx, rows, ewv, acc,
             semi0, semi1, semg0, semg1, seme0, seme1) = rest
        else:
            in_hbm = None
            (out_hbm, sidx, didx, rows, ewv, acc,
             semi0, semi1, semg0, semg1, seme0, seme1) = rest
        semi = (semi0, semi1)
        semg = (semg0, semg1)
        seme = (seme0, seme1)
        cid = lax.axis_index("c")
        sid = lax.axis_index("s")
        wid = sid * _NCORES + cid
        ebase = wid * epw

        def idx_start(ch, b):
            base = ebase + ch * _C
            pltpu.async_copy(src_hbm.at[pl.ds(base, _C)], sidx.at[b], semi[b])
            pltpu.async_copy(dst_hbm.at[pl.ds(base, _C)], didx.at[b], semi[b])

        def idx_wait(ch, b):
            base = ebase + ch * _C
            pltpu.make_async_copy(src_hbm.at[pl.ds(base, _C)], sidx.at[b], semi[b]).wait()
            pltpu.make_async_copy(dst_hbm.at[pl.ds(base, _C)], didx.at[b], semi[b]).wait()

        def fetch_start(ch, b):
            base = ebase + ch * _C
            pltpu.async_copy(x_hbm.at[sidx.at[b]], rows.at[b], semg[b])
            pltpu.async_copy(ew_hbm.at[pl.ds(base, _C)], ewv.at[b], seme[b])

        def fetch_wait(ch, b):
            base = ebase + ch * _C
            pltpu.make_async_copy(x_hbm.at[sidx.at[b]], rows.at[b], semg[b]).wait()
            pltpu.make_async_copy(ew_hbm.at[pl.ds(base, _C)], ewv.at[b], seme[b]).wait()

        nzch = _N // _C

        if agg_in is None:
            rows_z = rows.at[0]

            @pl.loop(0, _C)
            def _(i):
                for g in range(_D // 16):
                    rows_z[i, pl.ds(g * 16, 16)] = jnp.zeros((16,), jnp.float32)

            @pl.loop(0, (nzch + _NSUB - 1) // _NSUB)
            def _(t):
                j = sid + t * _NSUB

                @pl.when(j < nzch)
                def _():
                    pltpu.sync_copy(rows_z, acc.at[pl.ds(j * _C, _C)])
        else:
            @pl.loop(0, (_NCH_N + _NSUB - 1) // _NSUB)
            def _(t):
                j = sid + t * _NSUB

                @pl.when(j < _NCH_N)
                def _():
                    pltpu.sync_copy(in_hbm.at[cid].at[pl.ds(j * _CR, _CR)],
                                    acc.at[pl.ds(j * _CR, _CR)])

        plsc.subcore_barrier()

        idx_start(0, 0)
        idx_start(1, 1)
        idx_wait(0, 0)
        fetch_start(0, 0)

        @pl.loop(0, (nchunk + 1) // 2)
        def _(p):
            for kk in range(2):
                ch = p * 2 + kk
                b = kk
                nb = 1 - kk

                @pl.when(ch < nchunk)
                def _():
                    @pl.when(ch + 1 < nchunk)
                    def _():
                        idx_wait(ch + 1, nb)
                        fetch_start(ch + 1, nb)

                    fetch_wait(ch, b)
                    rows_b = rows.at[b]
                    ewv_b = ewv.at[b]

                    @pl.loop(0, _C, step=4)
                    def _(r0):
                        for dr in range(4):
                            for g in range(_D // 16):
                                s = pl.ds(g * 16, 16)
                                rows_b[r0 + dr, s] = rows_b[r0 + dr, s] * ewv_b[r0 + dr, s]

                    pltpu.sync_copy(rows_b, acc.at[didx.at[b]], add=True)

                    @pl.when(ch + 2 < nchunk)
                    def _():
                        idx_start(ch + 2, b)

        plsc.subcore_barrier()

        @pl.loop(0, (_NCH_N + _NSUB - 1) // _NSUB)
        def _(t):
            j = sid + t * _NSUB

            @pl.when(j < _NCH_N)
            def _():
                pltpu.sync_copy(acc.at[pl.ds(j * _CR, _CR)],
                                out_hbm.at[cid].at[pl.ds(j * _CR, _CR)])

    if agg_in is not None:
        return k(x, src, dst, ew, agg_in)
    return k(x, src, dst, ew)


def _final_body(nf_ref, na_ref, agg_ref, wl2_ref, wsc_ref, o_ref):
    agg = agg_ref[0] + agg_ref[1]
    y = jnp.dot(agg, wl2_ref[...], preferred_element_type=jnp.float32)
    nf = nf_ref[...]
    for v in range(_NA):
        y = y + jnp.dot(nf, wsc_ref[v], preferred_element_type=jnp.float32) * na_ref[:, v:v + 1]
    o_ref[...] = nf + _silu_n(y)


def _final(nf, na, agg2, wl2, wsc):
    blk = 2000
    return pl.pallas_call(
        _final_body,
        grid=(_N // blk,),
        in_specs=[
            pl.BlockSpec((blk, _D), lambda i: (i, 0)),
            pl.BlockSpec((blk, _NA), lambda i: (i, 0)),
            pl.BlockSpec((_NCORES, blk, _D), lambda i: (0, i, 0)),
            pl.BlockSpec((_D, _D), lambda i: (0, 0)),
            pl.BlockSpec((_NA, _D, _D), lambda i: (0, 0, 0)),
        ],
        out_specs=pl.BlockSpec((blk, _D), lambda i: (i, 0)),
        out_shape=jax.ShapeDtypeStruct((_N, _D), jnp.float32),
    )(nf, na, agg2, wl2, wsc)


def kernel(node_features, edge_index, edge_attrs, edge_embeddings, node_attrs,
           W_lin1, W_fc0, W_fc1, W_fc2, W_lin2, W_sc):
    src = edge_index[0]
    dst = edge_index[1]

    w0 = W_fc0 / math.sqrt(_NB)
    w1 = W_fc1 / math.sqrt(_H)
    w2 = W_fc2 / math.sqrt(_H)

    x = _lin1(node_features, W_lin1 / math.sqrt(_D))

    agg2 = None
    lo = 0
    for n_edges in _SLABS:
        sl = slice(lo, lo + n_edges)
        lo += n_edges
        ew = _edge_mlp(edge_embeddings[sl], edge_attrs[sl], w0, w1, w2)
        agg2 = _sc_message_pass(x, src[sl], dst[sl], ew, agg2)

    wl2 = W_lin2 / (math.sqrt(_D) * math.sqrt(_AVG_NEIGH))
    wsc = jnp.transpose(W_sc, (1, 0, 2)) / math.sqrt(_D * _NA)
    return _final(node_features, node_attrs, agg2, wl2, wsc)
